# Initial kernel scaffold; baseline (speedup 1.0000x reference)
import jax
import jax.numpy as jnp
from jax.experimental import pallas as pl


def kernel(env_length, edge_length, edge_index, env_index, env_radial, edge_radial, node_emb, env_hidden, edge_hidden, W_emb, b_emb, hW1, hb1, hW2, hb2, rW1, rb1, rW2, rb2, node_ln_g, node_ln_b, edge_ln_g, edge_ln_b):
    raise NotImplementedError("write your pallas kernel here")


# speedup vs baseline: 1.6222x; 1.6222x over previous
import functools

import jax
import jax.numpy as jnp
from jax import lax
from jax.experimental import pallas as pl
from jax.experimental.pallas import tpu as pltpu
from jax.experimental.pallas import tpu_sc as plsc

N = 50000
D = 32
RC = 5.0
NC, NS = 2, 16
NW = NC * NS
NPAD = 51200
ROWS_PT = NPAD // NS
CH = 128
SUP = 1024
SUP_S = 512
CD = 16
A_OLD = 0.89442719
A_NEW = 0.4472


def _sc_mesh():
    return plsc.VectorSubcoreMesh(core_axis_name="c", subcore_axis_name="s",
                                  num_cores=NC, num_subcores=NS)


_SC_PARAMS = pltpu.CompilerParams(use_tc_tiling_on_sc=False)


def _seg_sum_partials(radial_pad, idx2d):
    EP = radial_pad.shape[0]
    epw = EP // NW
    nsup = epw // SUP_S
    rows_per_sup = SUP_S // CH

    @functools.partial(
        pl.kernel,
        out_type=jax.ShapeDtypeStruct((NC, NPAD, D), jnp.float32),
        mesh=_sc_mesh(),
        compiler_params=_SC_PARAMS,
        scratch_types=[
            pltpu.VMEM((rows_per_sup, CH), jnp.int32),
            pltpu.VMEM((SUP_S, D), jnp.float32),
            pltpu.VMEM_SHARED((NPAD, D), jnp.float32),
        ],
    )
    def k(radial_hbm, idx_hbm, out_hbm, idxv, rows, acc):
        cid = lax.axis_index("c")
        sid = lax.axis_index("s")
        wid = cid * NS + sid
        zero16 = jnp.zeros((16,), jnp.float32)

        def _zb(i, carry):
            rows[i, pl.ds(0, 16)] = zero16
            rows[i, pl.ds(16, 16)] = zero16
            return carry
        lax.fori_loop(0, CH, _zb, 0)

        def _za(i, carry):
            pltpu.sync_copy(rows.at[pl.ds(0, CH)],
                            acc.at[pl.ds(sid * ROWS_PT + i * CH, CH)])
            return carry
        lax.fori_loop(0, ROWS_PT // CH, _za, 0)
        plsc.subcore_barrier()

        base = wid * epw
        irow_base = wid * (epw // CH)

        def _sup(t, carry):
            pltpu.sync_copy(idx_hbm.at[pl.ds(irow_base + t * rows_per_sup,
                                             rows_per_sup)], idxv)
            pltpu.sync_copy(radial_hbm.at[pl.ds(base + t * SUP_S, SUP_S)], rows)
            for j in range(rows_per_sup):
                pltpu.sync_copy(rows.at[pl.ds(j * CH, CH)],
                                acc.at[idxv.at[j]], add=True)
            return carry
        lax.fori_loop(0, nsup, _sup, 0)
        plsc.subcore_barrier()

        pltpu.sync_copy(acc.at[pl.ds(sid * ROWS_PT, ROWS_PT)],
                        out_hbm.at[cid, pl.ds(sid * ROWS_PT, ROWS_PT)])

    return k(radial_pad, idx2d)


def _seg_cnt_partials(idx2d):
    nrows = idx2d.shape[0]
    EP = nrows * CH
    epw = EP // NW
    nsup = epw // SUP_S
    rows_per_sup = SUP_S // CH

    @functools.partial(
        pl.kernel,
        out_type=jax.ShapeDtypeStruct((NC, NPAD, CD), jnp.float32),
        mesh=_sc_mesh(),
        compiler_params=_SC_PARAMS,
        scratch_types=[
            pltpu.VMEM((rows_per_sup, CH), jnp.int32),
            pltpu.VMEM((CH, CD), jnp.float32),
            pltpu.VMEM((CH, CD), jnp.float32),
            pltpu.VMEM_SHARED((NPAD, CD), jnp.float32),
        ],
    )
    def k(idx_hbm, out_hbm, idxv, ones_b, zbuf, acc):
        cid = lax.axis_index("c")
        sid = lax.axis_index("s")
        wid = cid * NS + sid
        zero16 = jnp.zeros((16,), jnp.float32)
        one16 = jnp.ones((16,), jnp.float32)

        def _fill(i, carry):
            ones_b[i, pl.ds(0, 16)] = one16
            zbuf[i, pl.ds(0, 16)] = zero16
            return carry
        lax.fori_loop(0, CH, _fill, 0)

        def _za(i, carry):
            pltpu.sync_copy(zbuf, acc.at[pl.ds(sid * ROWS_PT + i * CH, CH)])
            return carry
        lax.fori_loop(0, ROWS_PT // CH, _za, 0)
        plsc.subcore_barrier()

        irow_base = wid * (epw // CH)

        def _sup(t, carry):
            pltpu.sync_copy(idx_hbm.at[pl.ds(irow_base + t * rows_per_sup,
                                             rows_per_sup)], idxv)
            for j in range(rows_per_sup):
                pltpu.sync_copy(ones_b, acc.at[idxv.at[j]], add=True)
            return carry
        lax.fori_loop(0, nsup, _sup, 0)
        plsc.subcore_barrier()

        pltpu.sync_copy(acc.at[pl.ds(sid * ROWS_PT, ROWS_PT)],
                        out_hbm.at[cid, pl.ds(sid * ROWS_PT, ROWS_PT)])

    return k(idx2d)


def _gather_rows(table, idx2d):
    nrows = idx2d.shape[0]
    EP = nrows * CH
    epw = EP // NW
    nsup = epw // SUP
    rows_per_sup = SUP // CH

    @functools.partial(
        pl.kernel,
        out_type=jax.ShapeDtypeStruct((EP, D), jnp.float32),
        mesh=_sc_mesh(),
        compiler_params=_SC_PARAMS,
        scratch_types=[
            pltpu.VMEM((rows_per_sup, CH), jnp.int32),
            pltpu.VMEM((SUP, D), jnp.float32),
            pltpu.SemaphoreType.DMA,
        ],
    )
    def k(table_hbm, idx_hbm, out_hbm, idxv, rows, sem):
        cid = lax.axis_index("c")
        sid = lax.axis_index("s")
        wid = cid * NS + sid
        base = wid * epw
        irow_base = wid * (epw // CH)

        def _sup(t, carry):
            pltpu.sync_copy(idx_hbm.at[pl.ds(irow_base + t * rows_per_sup,
                                             rows_per_sup)], idxv)
            for j in range(rows_per_sup):
                pltpu.async_copy(table_hbm.at[idxv.at[j]],
                                 rows.at[pl.ds(j * CH, CH)], sem).wait()
            pltpu.sync_copy(rows, out_hbm.at[pl.ds(base + t * SUP, SUP)])
            return carry
        lax.fori_loop(0, nsup, _sup, 0)

    return k(table, idx2d)


def _node_body(sref, cref, nref, wref, bref, gref, b2ref, oref):
    s = sref[0] + sref[1]
    c = cref[0, :, 0:1] + cref[1, :, 0:1]
    w = jnp.dot(s, wref[...], preferred_element_type=jnp.float32) + c * bref[...]
    ne = nref[...]
    agg = ne * (w / jnp.maximum(c, 1.0))
    m = jnp.mean(agg, axis=-1, keepdims=True)
    v = jnp.mean((agg - m) ** 2, axis=-1, keepdims=True)
    ln = (agg - m) * lax.rsqrt(v + 1e-5) * gref[...] + b2ref[...]
    oref[...] = A_OLD * ne + A_NEW * ln


def _node_update(sum_p, cnt_p, node_emb, W_emb, b_emb2, ln_g2, ln_b2):
    BN = 2000
    grid = (N // BN,)
    return pl.pallas_call(
        _node_body,
        grid=grid,
        in_specs=[
            pl.BlockSpec((NC, BN, D), lambda i: (0, i, 0)),
            pl.BlockSpec((NC, BN, CD), lambda i: (0, i, 0)),
            pl.BlockSpec((BN, D), lambda i: (i, 0)),
            pl.BlockSpec((D, D), lambda i: (0, 0)),
            pl.BlockSpec((1, D), lambda i: (0, 0)),
            pl.BlockSpec((1, D), lambda i: (0, 0)),
            pl.BlockSpec((1, D), lambda i: (0, 0)),
        ],
        out_specs=pl.BlockSpec((BN, D), lambda i: (i, 0)),
        out_shape=jax.ShapeDtypeStruct((N, D), jnp.float32),
    )(sum_p, cnt_p, node_emb, W_emb, b_emb2, ln_g2, ln_b2)


def _branch_body(gref, href, rref, lref,
                 hW1a, hW1b, hb1, hW2, hb2,
                 rW1a, rW1b, rb1, rW2, rb2,
                 lng, lnb, oh, orad):
    g = gref[...]
    h = href[...]
    r = rref[...]
    x = (jnp.dot(g, hW1a[...], preferred_element_type=jnp.float32)
         + jnp.dot(h, hW1b[...], preferred_element_type=jnp.float32)
         + hb1[...])
    h1 = x / (1.0 + jnp.exp(-x))
    hn = jnp.dot(h1, hW2[...], preferred_element_type=jnp.float32) + hb2[...]
    oh[...] = hn
    y = (jnp.dot(r, rW1a[...], preferred_element_type=jnp.float32)
         + jnp.dot(hn, rW1b[...], preferred_element_type=jnp.float32)
         + rb1[...])
    r1 = y / (1.0 + jnp.exp(-y))
    r2 = jnp.dot(r1, rW2[...], preferred_element_type=jnp.float32) + rb2[...]
    m = jnp.mean(r2, axis=-1, keepdims=True)
    v = jnp.mean((r2 - m) ** 2, axis=-1, keepdims=True)
    ln = (r2 - m) * lax.rsqrt(v + 1e-5) * lng[...] + lnb[...]
    xl = lref[...]
    xr = xl * (1.0 / RC)
    xr2 = xr * xr
    xr3 = xr2 * xr
    xr6 = xr3 * xr3
    ud = (1.0 - 28.0 * xr6 + 48.0 * xr6 * xr - 21.0 * xr6 * xr2)
    ud = ud * (xl < RC).astype(jnp.float32)
    orad[...] = A_OLD * r + A_NEW * ud * ln


def _branch_update(gathered, hidden, radial, length2, wts, E):
    BN = 2000
    grid = (E // BN,)
    row = lambda i: (i, 0)
    full = lambda i: (0, 0)
    wspecs = [
        pl.BlockSpec((D, 64), full), pl.BlockSpec((D, 64), full),
        pl.BlockSpec((1, 64), full), pl.BlockSpec((64, D), full),
        pl.BlockSpec((1, D), full),
        pl.BlockSpec((D, 64), full), pl.BlockSpec((D, 64), full),
        pl.BlockSpec((1, 64), full), pl.BlockSpec((64, D), full),
        pl.BlockSpec((1, D), full),
        pl.BlockSpec((1, D), full), pl.BlockSpec((1, D), full),
    ]
    return pl.pallas_call(
        _branch_body,
        grid=grid,
        in_specs=[
            pl.BlockSpec((BN, D), row),
            pl.BlockSpec((BN, D), row),
            pl.BlockSpec((BN, D), row),
            pl.BlockSpec((BN, 1), row),
        ] + wspecs,
        out_specs=[pl.BlockSpec((BN, D), row), pl.BlockSpec((BN, D), row)],
        out_shape=[jax.ShapeDtypeStruct((E, D), jnp.float32),
                   jax.ShapeDtypeStruct((E, D), jnp.float32)],
    )(gathered, hidden, radial, length2, *wts)


def _pad_to(x, total, value):
    n = x.shape[0]
    if total == n:
        return x
    pad = jnp.full((total - n,) + x.shape[1:], value, x.dtype)
    return jnp.concatenate([x, pad], axis=0)


def kernel(env_length, edge_length, edge_index, env_index, env_radial,
           edge_radial, node_emb, env_hidden, edge_hidden,
           W_emb, b_emb, hW1, hb1, hW2, hb2, rW1, rb1, rW2, rb2,
           node_ln_g, node_ln_b, edge_ln_g, edge_ln_b):
    E_env = env_radial.shape[0]
    E_edge = edge_radial.shape[0]
    unit_s = NW * SUP_S
    unit_g = NW * SUP
    EPs_env = ((E_env + unit_s - 1) // unit_s) * unit_s
    EPg_env = ((E_env + unit_g - 1) // unit_g) * unit_g
    EPg_edge = ((E_edge + unit_g - 1) // unit_g) * unit_g

    dst = env_index[1].astype(jnp.int32)
    src_env = env_index[0].astype(jnp.int32)
    src_edge = edge_index[0].astype(jnp.int32)

    dst2d = _pad_to(dst, EPs_env, N).reshape(EPs_env // CH, CH)
    rad_pad = _pad_to(env_radial, EPs_env, 0.0)

    sum_p = _seg_sum_partials(rad_pad, dst2d)
    cnt_p = _seg_cnt_partials(dst2d)

    node_emb_new = _node_update(
        sum_p, cnt_p, node_emb, W_emb,
        b_emb.reshape(1, D), node_ln_g.reshape(1, D), node_ln_b.reshape(1, D))

    gidx_env = _pad_to(src_env, EPg_env, 0).reshape(EPg_env // CH, CH)
    gidx_edge = _pad_to(src_edge, EPg_edge, 0).reshape(EPg_edge // CH, CH)
    g_env = _gather_rows(node_emb_new, gidx_env)
    g_edge = _gather_rows(node_emb_new, gidx_edge)

    wts = [hW1[:D], hW1[D:], hb1.reshape(1, 64), hW2, hb2.reshape(1, D),
           rW1[:D], rW1[D:], rb1.reshape(1, 64), rW2, rb2.reshape(1, D),
           edge_ln_g.reshape(1, D), edge_ln_b.reshape(1, D)]

    env_hidden_new, env_radial_new = _branch_update(
        g_env, env_hidden, env_radial, env_length.reshape(E_env, 1), wts, E_env)
    edge_hidden_new, edge_radial_new = _branch_update(
        g_edge, edge_hidden, edge_radial, edge_length.reshape(E_edge, 1), wts,
        E_edge)

    return (env_radial_new, env_hidden_new, edge_radial_new, edge_hidden_new,
            node_emb_new)


# speedup vs baseline: 4.5975x; 2.8340x over previous
import functools

import jax
import jax.numpy as jnp
from jax import lax
from jax.experimental import pallas as pl
from jax.experimental.pallas import tpu as pltpu
from jax.experimental.pallas import tpu_sc as plsc

N = 50000
D = 32
RC = 5.0
NC, NS = 2, 16
NW = NC * NS
NPAD = 51200
ROWS_PT = NPAD // NS
CH = 128
SUP = 256
CD = 16
BN = 3200
ST = BN // 4
A_OLD = 0.89442719
A_NEW = 0.4472


def _sc_mesh():
    return plsc.VectorSubcoreMesh(core_axis_name="c", subcore_axis_name="s",
                                  num_cores=NC, num_subcores=NS)


_SC_PARAMS = pltpu.CompilerParams(use_tc_tiling_on_sc=False)


def _stripe_perm(a):
    return a.reshape(-1, 4, ST).swapaxes(1, 2).reshape(-1)


def _seg_sum_partials(rows_src, idx2d, total_chunks):
    rows_per_chunk = SUP // CH

    @functools.partial(
        pl.kernel,
        out_type=jax.ShapeDtypeStruct((NC, NPAD, D), jnp.float32),
        mesh=_sc_mesh(),
        compiler_params=_SC_PARAMS,
        scratch_types=[
            pltpu.VMEM((rows_per_chunk, CH), jnp.int32),
            pltpu.VMEM((SUP, D), jnp.float32),
            pltpu.VMEM_SHARED((NPAD, D), jnp.float32),
        ],
    )
    def k(rows_hbm, idx_hbm, out_hbm, idxv, rows, acc):
        cid = lax.axis_index("c")
        sid = lax.axis_index("s")
        wid = cid * NS + sid
        zero16 = jnp.zeros((16,), jnp.float32)

        def _zb(i, carry):
            rows[i, pl.ds(0, 16)] = zero16
            rows[i, pl.ds(16, 16)] = zero16
            return carry
        lax.fori_loop(0, CH, _zb, 0)

        def _za(i, carry):
            pltpu.sync_copy(rows.at[pl.ds(0, CH)],
                            acc.at[pl.ds(sid * ROWS_PT + i * CH, CH)])
            return carry
        lax.fori_loop(0, ROWS_PT // CH, _za, 0)
        plsc.subcore_barrier()

        nchunks = (total_chunks - wid + NW - 1) // NW

        def _chunk(i, carry):
            t = wid + i * NW
            pltpu.sync_copy(idx_hbm.at[pl.ds(t * rows_per_chunk,
                                             rows_per_chunk)], idxv)
            pltpu.sync_copy(rows_hbm.at[pl.ds(t * SUP, SUP)], rows)
            for j in range(rows_per_chunk):
                pltpu.sync_copy(rows.at[pl.ds(j * CH, CH)],
                                acc.at[idxv.at[j]], add=True)
            return carry
        lax.fori_loop(0, nchunks, _chunk, 0)
        plsc.subcore_barrier()

        pltpu.sync_copy(acc.at[pl.ds(sid * ROWS_PT, ROWS_PT)],
                        out_hbm.at[cid, pl.ds(sid * ROWS_PT, ROWS_PT)])

    return k(rows_src, idx2d)


def _seg_cnt_partials(idx2d, total_chunks):
    rows_per_chunk = SUP // CH

    @functools.partial(
        pl.kernel,
        out_type=jax.ShapeDtypeStruct((NC, NPAD, CD), jnp.float32),
        mesh=_sc_mesh(),
        compiler_params=_SC_PARAMS,
        scratch_types=[
            pltpu.VMEM((rows_per_chunk, CH), jnp.int32),
            pltpu.VMEM((CH, CD), jnp.float32),
            pltpu.VMEM((CH, CD), jnp.float32),
            pltpu.VMEM_SHARED((NPAD, CD), jnp.float32),
        ],
    )
    def k(idx_hbm, out_hbm, idxv, ones_b, zbuf, acc):
        cid = lax.axis_index("c")
        sid = lax.axis_index("s")
        wid = cid * NS + sid
        zero16 = jnp.zeros((16,), jnp.float32)
        one16 = jnp.ones((16,), jnp.float32)

        def _fill(i, carry):
            ones_b[i, pl.ds(0, 16)] = one16
            zbuf[i, pl.ds(0, 16)] = zero16
            return carry
        lax.fori_loop(0, CH, _fill, 0)

        def _za(i, carry):
            pltpu.sync_copy(zbuf, acc.at[pl.ds(sid * ROWS_PT + i * CH, CH)])
            return carry
        lax.fori_loop(0, ROWS_PT // CH, _za, 0)
        plsc.subcore_barrier()

        nchunks = (total_chunks - wid + NW - 1) // NW

        def _chunk(i, carry):
            t = wid + i * NW
            pltpu.sync_copy(idx_hbm.at[pl.ds(t * rows_per_chunk,
                                             rows_per_chunk)], idxv)
            for j in range(rows_per_chunk):
                pltpu.sync_copy(ones_b, acc.at[idxv.at[j]], add=True)
            return carry
        lax.fori_loop(0, nchunks, _chunk, 0)
        plsc.subcore_barrier()

        pltpu.sync_copy(acc.at[pl.ds(sid * ROWS_PT, ROWS_PT)],
                        out_hbm.at[cid, pl.ds(sid * ROWS_PT, ROWS_PT)])

    return k(idx2d)


def _gather_rows(table, idx2d, total_chunks):
    EP = idx2d.shape[0] * CH
    rows_per_chunk = SUP // CH

    @functools.partial(
        pl.kernel,
        out_type=jax.ShapeDtypeStruct((EP, D), jnp.float32),
        mesh=_sc_mesh(),
        compiler_params=_SC_PARAMS,
        scratch_types=[
            pltpu.VMEM((rows_per_chunk, CH), jnp.int32),
            pltpu.VMEM((SUP, D), jnp.float32),
            pltpu.SemaphoreType.DMA,
        ],
    )
    def k(table_hbm, idx_hbm, out_hbm, idxv, rows, sem):
        cid = lax.axis_index("c")
        sid = lax.axis_index("s")
        wid = cid * NS + sid
        nchunks = (total_chunks - wid + NW - 1) // NW

        def _chunk(i, carry):
            t = wid + i * NW
            pltpu.sync_copy(idx_hbm.at[pl.ds(t * rows_per_chunk,
                                             rows_per_chunk)], idxv)
            d0 = pltpu.async_copy(table_hbm.at[idxv.at[0]],
                                  rows.at[pl.ds(0, CH)], sem)
            d1 = pltpu.async_copy(table_hbm.at[idxv.at[1]],
                                  rows.at[pl.ds(CH, CH)], sem)
            d0.wait()
            d1.wait()
            pltpu.sync_copy(rows, out_hbm.at[pl.ds(t * SUP, SUP)])
            return carry
        lax.fori_loop(0, nchunks, _chunk, 0)

    return k(table, idx2d)


def _repack_body(tref, oref):
    x = tref[...]
    parts = [jnp.swapaxes(x[:, r * ST:(r + 1) * ST], 0, 1) for r in range(4)]
    oref[...] = jnp.concatenate(parts, axis=1)


def _repack(radT, E):
    grid = (E // BN,)
    return pl.pallas_call(
        _repack_body,
        grid=grid,
        in_specs=[pl.BlockSpec((D, BN), lambda i: (0, i))],
        out_specs=pl.BlockSpec((ST, 4 * D), lambda i: (i, 0)),
        out_shape=jax.ShapeDtypeStruct((E // 4, 4 * D), jnp.float32),
    )(radT)


def _node_body(sref, cref, nref, wref, bref, gref, b2ref, oref):
    s = sref[0] + sref[1]
    c = cref[0, :, 0:1] + cref[1, :, 0:1]
    w = jnp.dot(s, wref[...], preferred_element_type=jnp.float32) + c * bref[...]
    ne = nref[...]
    agg = ne * (w / jnp.maximum(c, 1.0))
    m = jnp.mean(agg, axis=-1, keepdims=True)
    v = jnp.mean((agg - m) ** 2, axis=-1, keepdims=True)
    ln = (agg - m) * lax.rsqrt(v + 1e-5) * gref[...] + b2ref[...]
    oref[...] = A_OLD * ne + A_NEW * ln


def _node_update(sum_p, cnt_p, node_emb, W_emb, b_emb2, ln_g2, ln_b2):
    NBN = 2000
    grid = (N // NBN,)
    return pl.pallas_call(
        _node_body,
        grid=grid,
        in_specs=[
            pl.BlockSpec((NC, NBN, D), lambda i: (0, i, 0)),
            pl.BlockSpec((NC, NBN, CD), lambda i: (0, i, 0)),
            pl.BlockSpec((NBN, D), lambda i: (i, 0)),
            pl.BlockSpec((D, D), lambda i: (0, 0)),
            pl.BlockSpec((1, D), lambda i: (0, 0)),
            pl.BlockSpec((1, D), lambda i: (0, 0)),
            pl.BlockSpec((1, D), lambda i: (0, 0)),
        ],
        out_specs=pl.BlockSpec((NBN, D), lambda i: (i, 0)),
        out_shape=jax.ShapeDtypeStruct((N, D), jnp.float32),
    )(sum_p, cnt_p, node_emb, W_emb, b_emb2, ln_g2, ln_b2)


def _tmul(a, b):
    return lax.dot_general(a, b, (((0,), (0,)), ((), ())),
                           preferred_element_type=jnp.float32)


def _branch_body(gref, href, rref, lref,
                 hW1a, hW1b, hb1, hW2, hb2,
                 rW1a, rW1b, rb1, rW2, rb2,
                 lng, lnb, oh, orad):
    p = gref[...]
    parts = [jnp.swapaxes(p[:, r * D:(r + 1) * D], 0, 1) for r in range(4)]
    gT = jnp.concatenate(parts, axis=1)
    hT = href[...]
    rT = rref[...]
    x = _tmul(hW1a[...], gT) + _tmul(hW1b[...], hT) + hb1[...]
    h1 = x / (1.0 + jnp.exp(-x))
    hn = _tmul(hW2[...], h1) + hb2[...]
    oh[...] = hn
    y = _tmul(rW1a[...], rT) + _tmul(rW1b[...], hn) + rb1[...]
    r1 = y / (1.0 + jnp.exp(-y))
    r2 = _tmul(rW2[...], r1) + rb2[...]
    m = jnp.mean(r2, axis=0, keepdims=True)
    v = jnp.mean((r2 - m) ** 2, axis=0, keepdims=True)
    ln = (r2 - m) * lax.rsqrt(v + 1e-5) * lng[...] + lnb[...]
    xl = lref[...]
    xr = xl * (1.0 / RC)
    xr2 = xr * xr
    xr3 = xr2 * xr
    xr6 = xr3 * xr3
    ud = (1.0 - 28.0 * xr6 + 48.0 * xr6 * xr - 21.0 * xr6 * xr2)
    ud = ud * (xl < RC).astype(jnp.float32)
    orad[...] = A_OLD * rT + A_NEW * ud * ln


def _branch_update(g_packed, hiddenT, radialT, lengthR, wts, E):
    grid = (E // BN,)
    colT = lambda i: (0, i)
    full = lambda i: (0, 0)
    wspecs = [
        pl.BlockSpec((D, 64), full), pl.BlockSpec((D, 64), full),
        pl.BlockSpec((64, 1), full), pl.BlockSpec((64, D), full),
        pl.BlockSpec((D, 1), full),
        pl.BlockSpec((D, 64), full), pl.BlockSpec((D, 64), full),
        pl.BlockSpec((64, 1), full), pl.BlockSpec((64, D), full),
        pl.BlockSpec((D, 1), full),
        pl.BlockSpec((D, 1), full), pl.BlockSpec((D, 1), full),
    ]
    return pl.pallas_call(
        _branch_body,
        grid=grid,
        in_specs=[
            pl.BlockSpec((ST, 4 * D), lambda i: (i, 0)),
            pl.BlockSpec((D, BN), colT),
            pl.BlockSpec((D, BN), colT),
            pl.BlockSpec((1, BN), colT),
        ] + wspecs,
        out_specs=[pl.BlockSpec((D, BN), colT), pl.BlockSpec((D, BN), colT)],
        out_shape=[jax.ShapeDtypeStruct((D, E), jnp.float32),
                   jax.ShapeDtypeStruct((D, E), jnp.float32)],
    )(g_packed, hiddenT, radialT, lengthR, *wts)


def kernel(env_length, edge_length, edge_index, env_index, env_radial,
           edge_radial, node_emb, env_hidden, edge_hidden,
           W_emb, b_emb, hW1, hb1, hW2, hb2, rW1, rb1, rW2, rb2,
           node_ln_g, node_ln_b, edge_ln_g, edge_ln_b):
    E_env = env_radial.shape[0]
    E_edge = edge_radial.shape[0]

    dst = env_index[1].astype(jnp.int32)
    src_env = env_index[0].astype(jnp.int32)
    src_edge = edge_index[0].astype(jnp.int32)

    rad_packed = _repack(jnp.transpose(env_radial), E_env)
    dst_perm = _stripe_perm(dst)
    dst2d = dst_perm.reshape(E_env // CH, CH)

    sum_p = _seg_sum_partials(rad_packed.reshape(E_env, D), dst2d,
                              E_env // SUP)
    cnt_p = _seg_cnt_partials(dst2d, E_env // SUP)

    node_emb_new = _node_update(
        sum_p, cnt_p, node_emb, W_emb,
        b_emb.reshape(1, D), node_ln_g.reshape(1, D), node_ln_b.reshape(1, D))

    EPg_edge = ((E_edge + SUP - 1) // SUP) * SUP
    gidx_env = _stripe_perm(src_env).reshape(E_env // CH, CH)
    gidx_edge = _stripe_perm(src_edge)
    gidx_edge = jnp.concatenate(
        [gidx_edge, jnp.zeros((EPg_edge - E_edge,), jnp.int32)]
    ).reshape(EPg_edge // CH, CH)
    g_env = _gather_rows(node_emb_new, gidx_env, E_env // SUP)
    g_edge = _gather_rows(node_emb_new, gidx_edge, EPg_edge // SUP)
    g_env_packed = g_env.reshape(E_env // 4, 4 * D)
    g_edge_packed = g_edge.reshape(EPg_edge // 4, 4 * D)

    wts = [hW1[:D], hW1[D:], hb1.reshape(64, 1), hW2,
           hb2.reshape(D, 1),
           rW1[:D], rW1[D:], rb1.reshape(64, 1), rW2,
           rb2.reshape(D, 1),
           edge_ln_g.reshape(D, 1), edge_ln_b.reshape(D, 1)]

    env_hidden_newT, env_radial_newT = _branch_update(
        g_env_packed, jnp.transpose(env_hidden), jnp.transpose(env_radial),
        env_length.reshape(1, E_env), wts, E_env)
    edge_hidden_newT, edge_radial_newT = _branch_update(
        g_edge_packed, jnp.transpose(edge_hidden), jnp.transpose(edge_radial),
        edge_length.reshape(1, E_edge), wts, E_edge)

    return (jnp.transpose(env_radial_newT), jnp.transpose(env_hidden_newT),
            jnp.transpose(edge_radial_newT), jnp.transpose(edge_hidden_newT),
            node_emb_new)


# speedup vs baseline: 4.8169x; 1.0477x over previous
import functools

import jax
import jax.numpy as jnp
from jax import lax
from jax.experimental import pallas as pl
from jax.experimental.pallas import tpu as pltpu
from jax.experimental.pallas import tpu_sc as plsc

N = 50000
D = 32
RC = 5.0
NC, NS = 2, 16
NW = NC * NS
NPAD = 51200
ROWS_PT = NPAD // NS
CH = 128
SUP = 256
CD = 16
BN = 3200
ST = BN // 4
A_OLD = 0.89442719
A_NEW = 0.4472


def _sc_mesh():
    return plsc.VectorSubcoreMesh(core_axis_name="c", subcore_axis_name="s",
                                  num_cores=NC, num_subcores=NS)


_SC_PARAMS = pltpu.CompilerParams(use_tc_tiling_on_sc=False)


def _stripe_perm(a):
    return a.reshape(-1, 4, ST).swapaxes(1, 2).reshape(-1)


def _seg_sum_partials(rows_src, idx2d, total_chunks):
    rows_per_chunk = SUP // CH

    @functools.partial(
        pl.kernel,
        out_type=jax.ShapeDtypeStruct((NC, NPAD, D), jnp.float32),
        mesh=_sc_mesh(),
        compiler_params=_SC_PARAMS,
        scratch_types=[
            pltpu.VMEM((rows_per_chunk, CH), jnp.int32),
            pltpu.VMEM((SUP, D), jnp.float32),
            pltpu.VMEM_SHARED((NPAD, D), jnp.float32),
        ],
    )
    def k(rows_hbm, idx_hbm, out_hbm, idxv, rows, acc):
        cid = lax.axis_index("c")
        sid = lax.axis_index("s")
        wid = cid * NS + sid
        zero16 = jnp.zeros((16,), jnp.float32)

        def _zb(i, carry):
            rows[i, pl.ds(0, 16)] = zero16
            rows[i, pl.ds(16, 16)] = zero16
            return carry
        lax.fori_loop(0, CH, _zb, 0)

        def _za(i, carry):
            pltpu.sync_copy(rows.at[pl.ds(0, CH)],
                            acc.at[pl.ds(sid * ROWS_PT + i * CH, CH)])
            return carry
        lax.fori_loop(0, ROWS_PT // CH, _za, 0)
        plsc.subcore_barrier()

        nchunks = (total_chunks - wid + NW - 1) // NW

        def _chunk(i, carry):
            t = wid + i * NW
            pltpu.sync_copy(idx_hbm.at[pl.ds(t * rows_per_chunk,
                                             rows_per_chunk)], idxv)
            pltpu.sync_copy(rows_hbm.at[pl.ds(t * SUP, SUP)], rows)
            for j in range(rows_per_chunk):
                pltpu.sync_copy(rows.at[pl.ds(j * CH, CH)],
                                acc.at[idxv.at[j]], add=True)
            return carry
        lax.fori_loop(0, nchunks, _chunk, 0)
        plsc.subcore_barrier()

        pltpu.sync_copy(acc.at[pl.ds(sid * ROWS_PT, ROWS_PT)],
                        out_hbm.at[cid, pl.ds(sid * ROWS_PT, ROWS_PT)])

    return k(rows_src, idx2d)


def _seg_cnt_partials(idx2d, total_chunks):
    rows_per_chunk = SUP // CH

    @functools.partial(
        pl.kernel,
        out_type=jax.ShapeDtypeStruct((NC, NPAD, CD), jnp.float32),
        mesh=_sc_mesh(),
        compiler_params=_SC_PARAMS,
        scratch_types=[
            pltpu.VMEM((rows_per_chunk, CH), jnp.int32),
            pltpu.VMEM((CH, CD), jnp.float32),
            pltpu.VMEM((CH, CD), jnp.float32),
            pltpu.VMEM_SHARED((NPAD, CD), jnp.float32),
        ],
    )
    def k(idx_hbm, out_hbm, idxv, ones_b, zbuf, acc):
        cid = lax.axis_index("c")
        sid = lax.axis_index("s")
        wid = cid * NS + sid
        zero16 = jnp.zeros((16,), jnp.float32)
        one16 = jnp.ones((16,), jnp.float32)

        def _fill(i, carry):
            ones_b[i, pl.ds(0, 16)] = one16
            zbuf[i, pl.ds(0, 16)] = zero16
            return carry
        lax.fori_loop(0, CH, _fill, 0)

        def _za(i, carry):
            pltpu.sync_copy(zbuf, acc.at[pl.ds(sid * ROWS_PT + i * CH, CH)])
            return carry
        lax.fori_loop(0, ROWS_PT // CH, _za, 0)
        plsc.subcore_barrier()

        nchunks = (total_chunks - wid + NW - 1) // NW

        def _chunk(i, carry):
            t = wid + i * NW
            pltpu.sync_copy(idx_hbm.at[pl.ds(t * rows_per_chunk,
                                             rows_per_chunk)], idxv)
            for j in range(rows_per_chunk):
                pltpu.sync_copy(ones_b, acc.at[idxv.at[j]], add=True)
            return carry
        lax.fori_loop(0, nchunks, _chunk, 0)
        plsc.subcore_barrier()

        pltpu.sync_copy(acc.at[pl.ds(sid * ROWS_PT, ROWS_PT)],
                        out_hbm.at[cid, pl.ds(sid * ROWS_PT, ROWS_PT)])

    return k(idx2d)


def _gather_rows(table, idx2d, total_chunks):
    EP = idx2d.shape[0] * CH
    rows_per_chunk = SUP // CH

    @functools.partial(
        pl.kernel,
        out_type=jax.ShapeDtypeStruct((EP, D), jnp.float32),
        mesh=_sc_mesh(),
        compiler_params=_SC_PARAMS,
        scratch_types=[
            pltpu.VMEM((rows_per_chunk, CH), jnp.int32),
            pltpu.VMEM((SUP, D), jnp.float32),
            pltpu.SemaphoreType.DMA,
        ],
    )
    def k(table_hbm, idx_hbm, out_hbm, idxv, rows, sem):
        cid = lax.axis_index("c")
        sid = lax.axis_index("s")
        wid = cid * NS + sid
        nchunks = (total_chunks - wid + NW - 1) // NW

        def _chunk(i, carry):
            t = wid + i * NW
            pltpu.sync_copy(idx_hbm.at[pl.ds(t * rows_per_chunk,
                                             rows_per_chunk)], idxv)
            d0 = pltpu.async_copy(table_hbm.at[idxv.at[0]],
                                  rows.at[pl.ds(0, CH)], sem)
            d1 = pltpu.async_copy(table_hbm.at[idxv.at[1]],
                                  rows.at[pl.ds(CH, CH)], sem)
            d0.wait()
            d1.wait()
            pltpu.sync_copy(rows, out_hbm.at[pl.ds(t * SUP, SUP)])
            return carry
        lax.fori_loop(0, nchunks, _chunk, 0)

    return k(table, idx2d)


def _eye(n):
    return (lax.broadcasted_iota(jnp.int32, (n, n), 0) ==
            lax.broadcasted_iota(jnp.int32, (n, n), 1)).astype(jnp.float32)


def _repack_body(tref, oref):
    x = tref[...]
    parts = [jnp.swapaxes(x[:, r * ST:(r + 1) * ST], 0, 1) for r in range(4)]
    oref[...] = jnp.concatenate(parts, axis=1)


def _repack(radT, E):
    grid = (E // BN,)
    return pl.pallas_call(
        _repack_body,
        grid=grid,
        in_specs=[pl.BlockSpec((D, BN), lambda i: (0, i))],
        out_specs=pl.BlockSpec((ST, 4 * D), lambda i: (i, 0)),
        out_shape=jax.ShapeDtypeStruct((E // 4, 4 * D), jnp.float32),
    )(radT)


def _node_body(sref, cref, nref, wref, bref, gref, b2ref, oref):
    s = sref[0] + sref[1]
    c = cref[0, :, 0:1] + cref[1, :, 0:1]
    w = jnp.dot(s, wref[...], preferred_element_type=jnp.float32) + c * bref[...]
    ne = nref[...]
    agg = ne * (w / jnp.maximum(c, 1.0))
    m = jnp.mean(agg, axis=-1, keepdims=True)
    v = jnp.mean((agg - m) ** 2, axis=-1, keepdims=True)
    ln = (agg - m) * lax.rsqrt(v + 1e-5) * gref[...] + b2ref[...]
    oref[...] = A_OLD * ne + A_NEW * ln


def _node_update(sum_p, cnt_p, node_emb, W_emb, b_emb2, ln_g2, ln_b2):
    NBN = 2000
    grid = (N // NBN,)
    return pl.pallas_call(
        _node_body,
        grid=grid,
        in_specs=[
            pl.BlockSpec((NC, NBN, D), lambda i: (0, i, 0)),
            pl.BlockSpec((NC, NBN, CD), lambda i: (0, i, 0)),
            pl.BlockSpec((NBN, D), lambda i: (i, 0)),
            pl.BlockSpec((D, D), lambda i: (0, 0)),
            pl.BlockSpec((1, D), lambda i: (0, 0)),
            pl.BlockSpec((1, D), lambda i: (0, 0)),
            pl.BlockSpec((1, D), lambda i: (0, 0)),
        ],
        out_specs=pl.BlockSpec((NBN, D), lambda i: (i, 0)),
        out_shape=jax.ShapeDtypeStruct((N, D), jnp.float32),
    )(sum_p, cnt_p, node_emb, W_emb, b_emb2, ln_g2, ln_b2)


def _tmul(a, b):
    return lax.dot_general(a, b, (((0,), (0,)), ((), ())),
                           preferred_element_type=jnp.float32)


def _tmul16(a, b):
    return lax.dot_general(a.astype(jnp.bfloat16), b.astype(jnp.bfloat16),
                           (((0,), (0,)), ((), ())),
                           preferred_element_type=jnp.float32)


def _branch_body(gref, href, rref, lref,
                 hW1a, hW1b, hb1, hW2, hb2,
                 rW1a, rW1b, rb1, rW2, rb2,
                 lng, lnb, oh, orad):
    p = gref[...]
    eye = _eye(D)
    parts = [lax.dot_general(eye, p[:, r * D:(r + 1) * D],
                             (((1,), (1,)), ((), ())),
                             preferred_element_type=jnp.float32)
             for r in range(4)]
    gT = jnp.concatenate(parts, axis=1)
    hT = href[...]
    rT = rref[...]
    x = _tmul16(hW1a[...], gT) + _tmul16(hW1b[...], hT) + hb1[...]
    h1 = x / (1.0 + jnp.exp(-x))
    hn = _tmul16(hW2[...], h1) + hb2[...]
    oh[...] = hn
    y = _tmul16(rW1a[...], rT) + _tmul16(rW1b[...], hn) + rb1[...]
    r1 = y / (1.0 + jnp.exp(-y))
    r2 = _tmul16(rW2[...], r1) + rb2[...]
    m = jnp.mean(r2, axis=0, keepdims=True)
    v = jnp.mean((r2 - m) ** 2, axis=0, keepdims=True)
    ln = (r2 - m) * lax.rsqrt(v + 1e-5) * lng[...] + lnb[...]
    xl = lref[...]
    xr = xl * (1.0 / RC)
    xr2 = xr * xr
    xr3 = xr2 * xr
    xr6 = xr3 * xr3
    ud = (1.0 - 28.0 * xr6 + 48.0 * xr6 * xr - 21.0 * xr6 * xr2)
    ud = ud * (xl < RC).astype(jnp.float32)
    orad[...] = A_OLD * rT + A_NEW * ud * ln


def _branch_update(g_packed, hiddenT, radialT, lengthR, wts, E):
    grid = (E // BN,)
    colT = lambda i: (0, i)
    full = lambda i: (0, 0)
    wspecs = [
        pl.BlockSpec((D, 64), full), pl.BlockSpec((D, 64), full),
        pl.BlockSpec((64, 1), full), pl.BlockSpec((64, D), full),
        pl.BlockSpec((D, 1), full),
        pl.BlockSpec((D, 64), full), pl.BlockSpec((D, 64), full),
        pl.BlockSpec((64, 1), full), pl.BlockSpec((64, D), full),
        pl.BlockSpec((D, 1), full),
        pl.BlockSpec((D, 1), full), pl.BlockSpec((D, 1), full),
    ]
    return pl.pallas_call(
        _branch_body,
        grid=grid,
        in_specs=[
            pl.BlockSpec((ST, 4 * D), lambda i: (i, 0)),
            pl.BlockSpec((D, BN), colT),
            pl.BlockSpec((D, BN), colT),
            pl.BlockSpec((1, BN), colT),
        ] + wspecs,
        out_specs=[pl.BlockSpec((D, BN), colT), pl.BlockSpec((D, BN), colT)],
        out_shape=[jax.ShapeDtypeStruct((D, E), jnp.float32),
                   jax.ShapeDtypeStruct((D, E), jnp.float32)],
    )(g_packed, hiddenT, radialT, lengthR, *wts)


def kernel(env_length, edge_length, edge_index, env_index, env_radial,
           edge_radial, node_emb, env_hidden, edge_hidden,
           W_emb, b_emb, hW1, hb1, hW2, hb2, rW1, rb1, rW2, rb2,
           node_ln_g, node_ln_b, edge_ln_g, edge_ln_b):
    E_env = env_radial.shape[0]
    E_edge = edge_radial.shape[0]

    dst = env_index[1].astype(jnp.int32)
    src_env = env_index[0].astype(jnp.int32)
    src_edge = edge_index[0].astype(jnp.int32)

    rad_packed = _repack(jnp.transpose(env_radial), E_env)
    dst_perm = _stripe_perm(dst)
    dst2d = dst_perm.reshape(E_env // CH, CH)

    sum_p = _seg_sum_partials(rad_packed.reshape(E_env, D), dst2d,
                              E_env // SUP)
    cnt_p = _seg_cnt_partials(dst2d, E_env // SUP)

    node_emb_new = _node_update(
        sum_p, cnt_p, node_emb, W_emb,
        b_emb.reshape(1, D), node_ln_g.reshape(1, D), node_ln_b.reshape(1, D))

    EPg_edge = ((E_edge + SUP - 1) // SUP) * SUP
    gidx_env = _stripe_perm(src_env).reshape(E_env // CH, CH)
    gidx_edge = _stripe_perm(src_edge)
    gidx_edge = jnp.concatenate(
        [gidx_edge, jnp.zeros((EPg_edge - E_edge,), jnp.int32)]
    ).reshape(EPg_edge // CH, CH)
    g_env = _gather_rows(node_emb_new, gidx_env, E_env // SUP)
    g_edge = _gather_rows(node_emb_new, gidx_edge, EPg_edge // SUP)
    g_env_packed = g_env.reshape(E_env // 4, 4 * D)
    g_edge_packed = g_edge.reshape(EPg_edge // 4, 4 * D)

    wts = [hW1[:D], hW1[D:], hb1.reshape(64, 1), hW2,
           hb2.reshape(D, 1),
           rW1[:D], rW1[D:], rb1.reshape(64, 1), rW2,
           rb2.reshape(D, 1),
           edge_ln_g.reshape(D, 1), edge_ln_b.reshape(D, 1)]

    env_hidden_newT, env_radial_newT = _branch_update(
        g_env_packed, jnp.transpose(env_hidden), jnp.transpose(env_radial),
        env_length.reshape(1, E_env), wts, E_env)
    edge_hidden_newT, edge_radial_newT = _branch_update(
        g_edge_packed, jnp.transpose(edge_hidden), jnp.transpose(edge_radial),
        edge_length.reshape(1, E_edge), wts, E_edge)

    return (jnp.transpose(env_radial_newT), jnp.transpose(env_hidden_newT),
            jnp.transpose(edge_radial_newT), jnp.transpose(edge_hidden_newT),
            node_emb_new)
